# baseline (speedup 1.0000x reference)
import functools

import jax
import jax.numpy as jnp
from jax import lax
from jax.experimental import pallas as pl
from jax.experimental.pallas import tpu as pltpu
from jax.experimental.pallas import tpu_sc as plsc

NLEARN = 40000
NCOURSE = 10000
EMB = 64
HALF = EMB // 2
NEDGE = 800000

LPAD = 40960
CPAD = 10240
PADL = 40480
PADC = 10120

NSUB = 16
CB = 128
NCHUNK = 396
EPAD = NSUB * NCHUNK * CB

LROWS = LPAD // NSUB
CROWS = CPAD // NSUB

_mesh = plsc.VectorSubcoreMesh(core_axis_name="c", subcore_axis_name="s")
_sc_params = pltpu.CompilerParams(use_tc_tiling_on_sc=False)


def _deg_body(src_hbm, dst_hbm, ones_hbm, zl_hbm, zc_hbm,
              degl_hbm, degc_hbm,
              accl_sh, accc_sh, idx_v0, idx_v1, ones_v, sd0, sd1):
  c = lax.axis_index("c")
  s = lax.axis_index("s")
  pltpu.sync_copy(ones_hbm, ones_v)

  @pl.when(c == 0)
  def _zero_l():
    pltpu.sync_copy(zl_hbm.at[pl.ds(s * LROWS, LROWS)],
                    accl_sh.at[pl.ds(s * LROWS, LROWS)])

  @pl.when(c == 1)
  def _zero_c():
    pltpu.sync_copy(zc_hbm.at[pl.ds(s * CROWS, CROWS)],
                    accc_sh.at[pl.ds(s * CROWS, CROWS)])

  plsc.subcore_barrier()

  ibufs = ((idx_v0, sd0), (idx_v1, sd1))

  def body(g, carry):
    for b in range(2):
      idx_v, sd = ibufs[b]
      j = 2 * g + b

      @pl.when(c == 0)
      def _hist_l():
        @pl.when(g >= 1)
        def _drain():
          pltpu.make_async_copy(ones_v, accl_sh.at[idx_v], sd).wait()
        pltpu.sync_copy(src_hbm.at[s, j], idx_v)
        pltpu.async_copy(ones_v, accl_sh.at[idx_v], sd, add=True)

      @pl.when(c == 1)
      def _hist_c():
        @pl.when(g >= 1)
        def _drain():
          pltpu.make_async_copy(ones_v, accc_sh.at[idx_v], sd).wait()
        pltpu.sync_copy(dst_hbm.at[s, j], idx_v)
        pltpu.async_copy(ones_v, accc_sh.at[idx_v], sd, add=True)

    return carry

  lax.fori_loop(0, NCHUNK // 2, body, 0)
  for b in range(2):
    idx_v, sd = ibufs[b]

    @pl.when(c == 0)
    def _drain_l():
      pltpu.make_async_copy(ones_v, accl_sh.at[idx_v], sd).wait()

    @pl.when(c == 1)
    def _drain_c():
      pltpu.make_async_copy(ones_v, accc_sh.at[idx_v], sd).wait()

  plsc.subcore_barrier()

  @pl.when(c == 0)
  def _out_l():
    pltpu.sync_copy(accl_sh.at[pl.ds(s * LROWS, LROWS)],
                    degl_hbm.at[pl.ds(s * LROWS, LROWS)])

  @pl.when(c == 1)
  def _out_c():
    pltpu.sync_copy(accc_sh.at[pl.ds(s * CROWS, CROWS)],
                    degc_hbm.at[pl.ds(s * CROWS, CROWS)])


_deg_call = pl.kernel(
    _deg_body,
    mesh=_mesh,
    out_type=[
        jax.ShapeDtypeStruct((LPAD, 16), jnp.float32),
        jax.ShapeDtypeStruct((CPAD, 16), jnp.float32),
    ],
    scratch_types=[
        pltpu.VMEM_SHARED((LPAD, 16), jnp.float32),
        pltpu.VMEM_SHARED((CPAD, 16), jnp.float32),
        pltpu.VMEM((CB,), jnp.int32),
        pltpu.VMEM((CB,), jnp.int32),
        pltpu.VMEM((CB, 16), jnp.float32),
        pltpu.SemaphoreType.DMA,
        pltpu.SemaphoreType.DMA,
    ],
    compiler_params=_sc_params,
)


def _agg_body(src_hbm, dst_hbm, xp_hbm, yp_hbm, zl_hbm, zc_hbm,
              outx_hbm, outy_hbm,
              accx_sh, accy_sh, *ring):
  c = lax.axis_index("c")
  s = lax.axis_index("s")
  offx = c * LPAD
  offy = c * CPAD

  bufs = tuple(ring[6 * b:6 * b + 6] + ring[18 + 4 * b:18 + 4 * b + 4]
               for b in range(3))

  pltpu.sync_copy(zl_hbm.at[pl.ds(s * LROWS, LROWS)],
                  accx_sh.at[pl.ds(s * LROWS, LROWS)])
  pltpu.sync_copy(zc_hbm.at[pl.ds(s * CROWS, CROWS)],
                  accy_sh.at[pl.ds(s * CROWS, CROWS)])
  plsc.subcore_barrier()

  def _prefetch(j, b):
    src_v, dst_v, srcg_v, dstg_v, xrows_v, yrows_v, gx, gy, _, _ = bufs[b]
    pltpu.sync_copy(src_hbm.at[s, j], src_v)
    pltpu.sync_copy(dst_hbm.at[s, j], dst_v)
    for r in range(CB // 16):
      sl = pl.ds(r * 16, 16)
      srcg_v[sl] = src_v[sl] + offx
      dstg_v[sl] = dst_v[sl] + offy
    pltpu.async_copy(xp_hbm.at[srcg_v], xrows_v, gx)
    pltpu.async_copy(yp_hbm.at[dstg_v], yrows_v, gy)

  def _drain_scatter(b):
    src_v, dst_v, _, _, xrows_v, yrows_v, _, _, sx, sy = bufs[b]
    pltpu.make_async_copy(yrows_v, accx_sh.at[src_v], sx).wait()
    pltpu.make_async_copy(xrows_v, accy_sh.at[dst_v], sy).wait()

  for b in range(2):
    _prefetch(b, b)

  def body(g, carry):
    for b in range(3):
      src_v, dst_v, srcg_v, dstg_v, xrows_v, yrows_v, gx, gy, sx, sy = bufs[b]
      j = 3 * g + b

      pltpu.make_async_copy(xp_hbm.at[srcg_v], xrows_v, gx).wait()
      pltpu.make_async_copy(yp_hbm.at[dstg_v], yrows_v, gy).wait()
      pltpu.async_copy(yrows_v, accx_sh.at[src_v], sx, add=True)
      pltpu.async_copy(xrows_v, accy_sh.at[dst_v], sy, add=True)

      @pl.when(j >= 1)
      def _drain():
        _drain_scatter((b + 2) % 3)

      @pl.when(j + 2 < NCHUNK)
      def _pref():
        _prefetch(j + 2, (b + 2) % 3)

    return carry

  lax.fori_loop(0, NCHUNK // 3, body, 0)
  _drain_scatter((NCHUNK - 1) % 3)
  plsc.subcore_barrier()

  pltpu.sync_copy(accx_sh.at[pl.ds(s * LROWS, LROWS)],
                  outx_hbm.at[pl.ds(offx + s * LROWS, LROWS)])
  pltpu.sync_copy(accy_sh.at[pl.ds(s * CROWS, CROWS)],
                  outy_hbm.at[pl.ds(offy + s * CROWS, CROWS)])


_agg_call = pl.kernel(
    _agg_body,
    mesh=_mesh,
    out_type=[
        jax.ShapeDtypeStruct((2 * LPAD, HALF), jnp.float32),
        jax.ShapeDtypeStruct((2 * CPAD, HALF), jnp.float32),
    ],
    scratch_types=(
        [
            pltpu.VMEM_SHARED((LPAD, HALF), jnp.float32),
            pltpu.VMEM_SHARED((CPAD, HALF), jnp.float32),
        ]
        + [
            pltpu.VMEM((CB,), jnp.int32),
            pltpu.VMEM((CB,), jnp.int32),
            pltpu.VMEM((CB,), jnp.int32),
            pltpu.VMEM((CB,), jnp.int32),
            pltpu.VMEM((CB, HALF), jnp.float32),
            pltpu.VMEM((CB, HALF), jnp.float32),
        ] * 3
        + [pltpu.SemaphoreType.DMA] * 12
    ),
    compiler_params=_sc_params,
)


BLK = 2048


def _scale0_body(nreal, deg_ref, feat_ref, isd_ref, featp_ref):
  i = pl.program_id(0)
  rows = lax.broadcasted_iota(jnp.int32, (BLK, 1), 0) + i * BLK
  mask = rows < nreal
  deg = deg_ref[:, 0:1]
  isd = 1.0 / (jnp.sqrt(deg) + 1e-8)
  isd = jnp.where(mask, isd, 0.0)
  isd_ref[...] = jnp.broadcast_to(isd, (BLK, 16))
  f = feat_ref[...]
  featp_ref[0] = jnp.where(mask, f[:, 0:HALF] * isd, 0.0)
  featp_ref[1] = jnp.where(mask, f[:, HALF:EMB] * isd, 0.0)


def _make_scale0(nreal, npad):
  grid = npad // BLK
  return pl.pallas_call(
      functools.partial(_scale0_body, nreal),
      grid=(grid,),
      in_specs=[
          pl.BlockSpec((BLK, 16), lambda i: (i, 0)),
          pl.BlockSpec((BLK, EMB), lambda i: (i, 0)),
      ],
      out_specs=[
          pl.BlockSpec((BLK, 16), lambda i: (i, 0)),
          pl.BlockSpec((2, BLK, HALF), lambda i: (0, i, 0)),
      ],
      out_shape=[
          jax.ShapeDtypeStruct((npad, 16), jnp.float32),
          jax.ShapeDtypeStruct((2, npad, HALF), jnp.float32),
      ],
  )


def _mid_body(acc_ref, isd_ref, f1_ref, fp1_ref):
  isd = isd_ref[:, 0:1]
  x0 = acc_ref[0] * isd
  x1 = acc_ref[1] * isd
  f1_ref[...] = jnp.concatenate([x0, x1], axis=1)
  fp1_ref[0] = x0 * isd
  fp1_ref[1] = x1 * isd


def _make_mid(npad):
  grid = npad // BLK
  return pl.pallas_call(
      _mid_body,
      grid=(grid,),
      in_specs=[
          pl.BlockSpec((2, BLK, HALF), lambda i: (0, i, 0)),
          pl.BlockSpec((BLK, 16), lambda i: (i, 0)),
      ],
      out_specs=[
          pl.BlockSpec((BLK, EMB), lambda i: (i, 0)),
          pl.BlockSpec((2, BLK, HALF), lambda i: (0, i, 0)),
      ],
      out_shape=[
          jax.ShapeDtypeStruct((npad, EMB), jnp.float32),
          jax.ShapeDtypeStruct((2, npad, HALF), jnp.float32),
      ],
  )


FBLK = 2000


def _final_body(f0_ref, f1_ref, acc_ref, isd_ref, out_ref):
  isd = isd_ref[:, 0:1]
  a = jnp.concatenate([acc_ref[0] * isd, acc_ref[1] * isd], axis=1)
  out_ref[...] = (f0_ref[...] + f1_ref[...] + a) * (1.0 / 3.0)


def _make_final(nreal):
  grid = nreal // FBLK
  return pl.pallas_call(
      _final_body,
      grid=(grid,),
      in_specs=[
          pl.BlockSpec((FBLK, EMB), lambda i: (i, 0)),
          pl.BlockSpec((FBLK, EMB), lambda i: (i, 0)),
          pl.BlockSpec((2, FBLK, HALF), lambda i: (0, i, 0)),
          pl.BlockSpec((FBLK, 16), lambda i: (i, 0)),
      ],
      out_specs=pl.BlockSpec((FBLK, EMB), lambda i: (i, 0)),
      out_shape=jax.ShapeDtypeStruct((nreal, EMB), jnp.float32),
  )


_scale0_l = _make_scale0(NLEARN, LPAD)
_scale0_c = _make_scale0(NCOURSE, CPAD)
_mid_l = _make_mid(LPAD)
_mid_c = _make_mid(CPAD)
_final_l = _make_final(NLEARN)
_final_c = _make_final(NCOURSE)


@jax.jit
def kernel(learners_feature, courses_feature, edge_src, edge_dst):
  src = jnp.concatenate(
      [edge_src, jnp.full((EPAD - NEDGE,), PADL, jnp.int32)]
  ).reshape(NSUB, NCHUNK, CB)
  dst = jnp.concatenate(
      [edge_dst, jnp.full((EPAD - NEDGE,), PADC, jnp.int32)]
  ).reshape(NSUB, NCHUNK, CB)

  ones16 = jnp.ones((CB, 16), jnp.float32)
  zl16 = jnp.zeros((LPAD, 16), jnp.float32)
  zc16 = jnp.zeros((CPAD, 16), jnp.float32)
  zl32 = jnp.zeros((LPAD, HALF), jnp.float32)
  zc32 = jnp.zeros((CPAD, HALF), jnp.float32)

  degl, degc = _deg_call(src, dst, ones16, zl16, zc16)
  isdl, xp0 = _scale0_l(degl, learners_feature)
  isdc, yp0 = _scale0_c(degc, courses_feature)

  ax1, ay1 = _agg_call(src, dst,
                       xp0.reshape(2 * LPAD, HALF),
                       yp0.reshape(2 * CPAD, HALF),
                       zl32, zc32)
  x1, xp1 = _mid_l(ax1.reshape(2, LPAD, HALF), isdl)
  y1, yp1 = _mid_c(ay1.reshape(2, CPAD, HALF), isdc)

  ax2, ay2 = _agg_call(src, dst,
                       xp1.reshape(2 * LPAD, HALF),
                       yp1.reshape(2 * CPAD, HALF),
                       zl32, zc32)

  learners_out = _final_l(learners_feature, x1,
                          ax2.reshape(2, LPAD, HALF), isdl)
  courses_out = _final_c(courses_feature, y1,
                         ay2.reshape(2, CPAD, HALF), isdc)
  return learners_out, courses_out


# speedup vs baseline: 1.0993x; 1.0993x over previous
import functools

import jax
import jax.numpy as jnp
from jax import lax
from jax.experimental import pallas as pl
from jax.experimental.pallas import tpu as pltpu
from jax.experimental.pallas import tpu_sc as plsc

NLEARN = 40000
NCOURSE = 10000
EMB = 64
HALF = EMB // 2
NEDGE = 800000

LPAD = 40960
CPAD = 10240
PADL = 40480
PADC = 10120

NSUB = 16
CB = 128
NCHUNK = 396
EPAD = NSUB * NCHUNK * CB

LROWS = LPAD // NSUB
CROWS = CPAD // NSUB

_mesh = plsc.VectorSubcoreMesh(core_axis_name="c", subcore_axis_name="s")
_sc_params = pltpu.CompilerParams(use_tc_tiling_on_sc=False)


def _deg_body(src_hbm, dst_hbm, ones_hbm, zl_hbm, zc_hbm,
              degl_hbm, degc_hbm,
              accl_sh, accc_sh, idx_v0, idx_v1, idx_v2, idx_v3, ones_v,
              si0, si1, si2, si3, sd0, sd1, sd2, sd3):
  c = lax.axis_index("c")
  s = lax.axis_index("s")
  pltpu.sync_copy(ones_hbm, ones_v)

  @pl.when(c == 0)
  def _zero_l():
    pltpu.sync_copy(zl_hbm.at[pl.ds(s * LROWS, LROWS)],
                    accl_sh.at[pl.ds(s * LROWS, LROWS)])

  @pl.when(c == 1)
  def _zero_c():
    pltpu.sync_copy(zc_hbm.at[pl.ds(s * CROWS, CROWS)],
                    accc_sh.at[pl.ds(s * CROWS, CROWS)])

  plsc.subcore_barrier()

  ibufs = ((idx_v0, si0, sd0), (idx_v1, si1, sd1),
           (idx_v2, si2, sd2), (idx_v3, si3, sd3))
  idx_hbm = (src_hbm, dst_hbm)
  acc_sh = (accl_sh, accc_sh)

  def _core(k):
    ihbm = idx_hbm[k]
    acc = acc_sh[k]

    for b in range(2):
      idx_v, si, _ = ibufs[b]
      pltpu.async_copy(ihbm.at[s, b], idx_v, si)

    def body(g, carry):
      for b in range(4):
        idx_v, si, sd = ibufs[b]
        j = 4 * g + b
        idx_p, si_p, sd_p = ibufs[(b + 2) % 4]

        @pl.when(j >= 2)
        def _drain():
          pltpu.make_async_copy(ones_v, acc.at[idx_p], sd_p).wait()

        @pl.when(j + 2 < NCHUNK)
        def _pref():
          pltpu.async_copy(ihbm.at[s, j + 2], idx_p, si_p)

        pltpu.make_async_copy(ihbm.at[s, j], idx_v, si).wait()
        pltpu.async_copy(ones_v, acc.at[idx_v], sd, add=True)
      return carry

    lax.fori_loop(0, NCHUNK // 4, body, 0)
    for b in (2, 3):
      idx_v, _, sd = ibufs[b]
      pltpu.make_async_copy(ones_v, acc.at[idx_v], sd).wait()

  @pl.when(c == 0)
  def _core_l():
    _core(0)

  @pl.when(c == 1)
  def _core_c():
    _core(1)

  plsc.subcore_barrier()

  @pl.when(c == 0)
  def _out_l():
    pltpu.sync_copy(accl_sh.at[pl.ds(s * LROWS, LROWS)],
                    degl_hbm.at[pl.ds(s * LROWS, LROWS)])

  @pl.when(c == 1)
  def _out_c():
    pltpu.sync_copy(accc_sh.at[pl.ds(s * CROWS, CROWS)],
                    degc_hbm.at[pl.ds(s * CROWS, CROWS)])


_deg_call = pl.kernel(
    _deg_body,
    mesh=_mesh,
    out_type=[
        jax.ShapeDtypeStruct((LPAD, 16), jnp.float32),
        jax.ShapeDtypeStruct((CPAD, 16), jnp.float32),
    ],
    scratch_types=[
        pltpu.VMEM_SHARED((LPAD, 16), jnp.float32),
        pltpu.VMEM_SHARED((CPAD, 16), jnp.float32),
        pltpu.VMEM((CB,), jnp.int32),
        pltpu.VMEM((CB,), jnp.int32),
        pltpu.VMEM((CB,), jnp.int32),
        pltpu.VMEM((CB,), jnp.int32),
        pltpu.VMEM((CB, 16), jnp.float32),
        pltpu.SemaphoreType.DMA,
        pltpu.SemaphoreType.DMA,
        pltpu.SemaphoreType.DMA,
        pltpu.SemaphoreType.DMA,
        pltpu.SemaphoreType.DMA,
        pltpu.SemaphoreType.DMA,
        pltpu.SemaphoreType.DMA,
        pltpu.SemaphoreType.DMA,
    ],
    compiler_params=_sc_params,
)


def _agg_body(src_hbm, dst_hbm, xp_hbm, yp_hbm, zl_hbm, zc_hbm,
              outx_hbm, outy_hbm,
              accx_sh, accy_sh, *ring):
  c = lax.axis_index("c")
  s = lax.axis_index("s")
  offx = c * LPAD
  offy = c * CPAD

  bufs = tuple(ring[6 * b:6 * b + 6] + ring[18 + 4 * b:18 + 4 * b + 4]
               for b in range(3))

  pltpu.sync_copy(zl_hbm.at[pl.ds(s * LROWS, LROWS)],
                  accx_sh.at[pl.ds(s * LROWS, LROWS)])
  pltpu.sync_copy(zc_hbm.at[pl.ds(s * CROWS, CROWS)],
                  accy_sh.at[pl.ds(s * CROWS, CROWS)])
  plsc.subcore_barrier()

  def _prefetch(j, b):
    src_v, dst_v, srcg_v, dstg_v, xrows_v, yrows_v, gx, gy, _, _ = bufs[b]
    pltpu.sync_copy(src_hbm.at[s, j], src_v)
    pltpu.sync_copy(dst_hbm.at[s, j], dst_v)
    for r in range(CB // 16):
      sl = pl.ds(r * 16, 16)
      srcg_v[sl] = src_v[sl] + offx
      dstg_v[sl] = dst_v[sl] + offy
    pltpu.async_copy(xp_hbm.at[srcg_v], xrows_v, gx)
    pltpu.async_copy(yp_hbm.at[dstg_v], yrows_v, gy)

  def _drain_scatter(b):
    src_v, dst_v, _, _, xrows_v, yrows_v, _, _, sx, sy = bufs[b]
    pltpu.make_async_copy(yrows_v, accx_sh.at[src_v], sx).wait()
    pltpu.make_async_copy(xrows_v, accy_sh.at[dst_v], sy).wait()

  for b in range(2):
    _prefetch(b, b)

  def body(g, carry):
    for b in range(3):
      src_v, dst_v, srcg_v, dstg_v, xrows_v, yrows_v, gx, gy, sx, sy = bufs[b]
      j = 3 * g + b

      pltpu.make_async_copy(xp_hbm.at[srcg_v], xrows_v, gx).wait()
      pltpu.make_async_copy(yp_hbm.at[dstg_v], yrows_v, gy).wait()
      pltpu.async_copy(yrows_v, accx_sh.at[src_v], sx, add=True)
      pltpu.async_copy(xrows_v, accy_sh.at[dst_v], sy, add=True)

      @pl.when(j >= 1)
      def _drain():
        _drain_scatter((b + 2) % 3)

      @pl.when(j + 2 < NCHUNK)
      def _pref():
        _prefetch(j + 2, (b + 2) % 3)

    return carry

  lax.fori_loop(0, NCHUNK // 3, body, 0)
  _drain_scatter((NCHUNK - 1) % 3)
  plsc.subcore_barrier()

  pltpu.sync_copy(accx_sh.at[pl.ds(s * LROWS, LROWS)],
                  outx_hbm.at[pl.ds(offx + s * LROWS, LROWS)])
  pltpu.sync_copy(accy_sh.at[pl.ds(s * CROWS, CROWS)],
                  outy_hbm.at[pl.ds(offy + s * CROWS, CROWS)])


_agg_call = pl.kernel(
    _agg_body,
    mesh=_mesh,
    out_type=[
        jax.ShapeDtypeStruct((2 * LPAD, HALF), jnp.float32),
        jax.ShapeDtypeStruct((2 * CPAD, HALF), jnp.float32),
    ],
    scratch_types=(
        [
            pltpu.VMEM_SHARED((LPAD, HALF), jnp.float32),
            pltpu.VMEM_SHARED((CPAD, HALF), jnp.float32),
        ]
        + [
            pltpu.VMEM((CB,), jnp.int32),
            pltpu.VMEM((CB,), jnp.int32),
            pltpu.VMEM((CB,), jnp.int32),
            pltpu.VMEM((CB,), jnp.int32),
            pltpu.VMEM((CB, HALF), jnp.float32),
            pltpu.VMEM((CB, HALF), jnp.float32),
        ] * 3
        + [pltpu.SemaphoreType.DMA] * 12
    ),
    compiler_params=_sc_params,
)


BLK = 2048


def _scale0_body(nreal, deg_ref, feat_ref, isd_ref, featp_ref):
  i = pl.program_id(0)
  rows = lax.broadcasted_iota(jnp.int32, (BLK, 1), 0) + i * BLK
  mask = rows < nreal
  deg = deg_ref[:, 0:1]
  isd = 1.0 / (jnp.sqrt(deg) + 1e-8)
  isd = jnp.where(mask, isd, 0.0)
  isd_ref[...] = jnp.broadcast_to(isd, (BLK, 16))
  f = feat_ref[...]
  featp_ref[0] = jnp.where(mask, f[:, 0:HALF] * isd, 0.0)
  featp_ref[1] = jnp.where(mask, f[:, HALF:EMB] * isd, 0.0)


def _make_scale0(nreal, npad):
  grid = npad // BLK
  return pl.pallas_call(
      functools.partial(_scale0_body, nreal),
      grid=(grid,),
      in_specs=[
          pl.BlockSpec((BLK, 16), lambda i: (i, 0)),
          pl.BlockSpec((BLK, EMB), lambda i: (i, 0)),
      ],
      out_specs=[
          pl.BlockSpec((BLK, 16), lambda i: (i, 0)),
          pl.BlockSpec((2, BLK, HALF), lambda i: (0, i, 0)),
      ],
      out_shape=[
          jax.ShapeDtypeStruct((npad, 16), jnp.float32),
          jax.ShapeDtypeStruct((2, npad, HALF), jnp.float32),
      ],
  )


def _mid_body(acc_ref, isd_ref, f1_ref, fp1_ref):
  isd = isd_ref[:, 0:1]
  x0 = acc_ref[0] * isd
  x1 = acc_ref[1] * isd
  f1_ref[...] = jnp.concatenate([x0, x1], axis=1)
  fp1_ref[0] = x0 * isd
  fp1_ref[1] = x1 * isd


def _make_mid(npad):
  grid = npad // BLK
  return pl.pallas_call(
      _mid_body,
      grid=(grid,),
      in_specs=[
          pl.BlockSpec((2, BLK, HALF), lambda i: (0, i, 0)),
          pl.BlockSpec((BLK, 16), lambda i: (i, 0)),
      ],
      out_specs=[
          pl.BlockSpec((BLK, EMB), lambda i: (i, 0)),
          pl.BlockSpec((2, BLK, HALF), lambda i: (0, i, 0)),
      ],
      out_shape=[
          jax.ShapeDtypeStruct((npad, EMB), jnp.float32),
          jax.ShapeDtypeStruct((2, npad, HALF), jnp.float32),
      ],
  )


FBLK = 2000


def _final_body(f0_ref, f1_ref, acc_ref, isd_ref, out_ref):
  isd = isd_ref[:, 0:1]
  a = jnp.concatenate([acc_ref[0] * isd, acc_ref[1] * isd], axis=1)
  out_ref[...] = (f0_ref[...] + f1_ref[...] + a) * (1.0 / 3.0)


def _make_final(nreal):
  grid = nreal // FBLK
  return pl.pallas_call(
      _final_body,
      grid=(grid,),
      in_specs=[
          pl.BlockSpec((FBLK, EMB), lambda i: (i, 0)),
          pl.BlockSpec((FBLK, EMB), lambda i: (i, 0)),
          pl.BlockSpec((2, FBLK, HALF), lambda i: (0, i, 0)),
          pl.BlockSpec((FBLK, 16), lambda i: (i, 0)),
      ],
      out_specs=pl.BlockSpec((FBLK, EMB), lambda i: (i, 0)),
      out_shape=jax.ShapeDtypeStruct((nreal, EMB), jnp.float32),
  )


_scale0_l = _make_scale0(NLEARN, LPAD)
_scale0_c = _make_scale0(NCOURSE, CPAD)
_mid_l = _make_mid(LPAD)
_mid_c = _make_mid(CPAD)
_final_l = _make_final(NLEARN)
_final_c = _make_final(NCOURSE)


@jax.jit
def kernel(learners_feature, courses_feature, edge_src, edge_dst):
  src = jnp.concatenate(
      [edge_src, jnp.full((EPAD - NEDGE,), PADL, jnp.int32)]
  ).reshape(NSUB, NCHUNK, CB)
  dst = jnp.concatenate(
      [edge_dst, jnp.full((EPAD - NEDGE,), PADC, jnp.int32)]
  ).reshape(NSUB, NCHUNK, CB)

  ones16 = jnp.ones((CB, 16), jnp.float32)
  zl16 = jnp.zeros((LPAD, 16), jnp.float32)
  zc16 = jnp.zeros((CPAD, 16), jnp.float32)
  zl32 = jnp.zeros((LPAD, HALF), jnp.float32)
  zc32 = jnp.zeros((CPAD, HALF), jnp.float32)

  degl, degc = _deg_call(src, dst, ones16, zl16, zc16)
  isdl, xp0 = _scale0_l(degl, learners_feature)
  isdc, yp0 = _scale0_c(degc, courses_feature)

  ax1, ay1 = _agg_call(src, dst,
                       xp0.reshape(2 * LPAD, HALF),
                       yp0.reshape(2 * CPAD, HALF),
                       zl32, zc32)
  x1, xp1 = _mid_l(ax1.reshape(2, LPAD, HALF), isdl)
  y1, yp1 = _mid_c(ay1.reshape(2, CPAD, HALF), isdc)

  ax2, ay2 = _agg_call(src, dst,
                       xp1.reshape(2 * LPAD, HALF),
                       yp1.reshape(2 * CPAD, HALF),
                       zl32, zc32)

  learners_out = _final_l(learners_feature, x1[:NLEARN],
                          ax2.reshape(2, LPAD, HALF)[:, :NLEARN],
                          isdl[:NLEARN])
  courses_out = _final_c(courses_feature, y1[:NCOURSE],
                         ay2.reshape(2, CPAD, HALF)[:, :NCOURSE],
                         isdc[:NCOURSE])
  return learners_out, courses_out
